# Initial kernel scaffold; baseline (speedup 1.0000x reference)
import jax
import jax.numpy as jnp
from jax.experimental import pallas as pl


def kernel(x, edge_index, W1, b1, W2, b2):
    raise NotImplementedError("write your pallas kernel here")


# speedup vs baseline: 2.5526x; 2.5526x over previous
import functools

import jax
import jax.numpy as jnp
from jax import lax
from jax.experimental import pallas as pl
from jax.experimental.pallas import tpu as pltpu
from jax.experimental.pallas import tpu_sc as plsc

_B, _C, _N, _K = 32, 384, 256, 16
_C_OUT = 768
_L = 16

def _sc_raw_body(x_hbm, idxt_hbm, out_hbm, table_v, idx_v, out_v):
    cid = lax.axis_index("c")
    sid = lax.axis_index("s")
    wid = sid * 2 + cid

    pltpu.sync_copy(x_hbm.at[pl.ds(wid * _N, _N)], table_v)
    pltpu.sync_copy(idxt_hbm.at[wid], idx_v)

    lane_iota = lax.iota(jnp.int32, _L)

    def group_body(g, _):
        col = g * _L
        rv = [idx_v[r, pl.ds(col, _L)] for r in range(_K)]

        def c_body(c, _):
            cvec = jnp.full((_L,), 0, jnp.int32) + c
            acc = plsc.load_gather(table_v, [rv[0], cvec])
            for r in range(1, _K):
                acc = jnp.maximum(acc, plsc.load_gather(table_v, [rv[r], cvec]))
            plsc.store_scatter(out_v, [lane_iota, cvec], acc)
            return ()

        lax.fori_loop(0, _C, c_body, ())
        pltpu.sync_copy(out_v, out_hbm.at[pl.ds(wid * _N + col, _L)])
        return ()

    lax.fori_loop(0, _N // _L, group_body, ())


@functools.cache
def _sc_gather_max():
    mesh = plsc.VectorSubcoreMesh(
        core_axis_name="c", subcore_axis_name="s",
        num_cores=2, num_subcores=16)
    return functools.partial(
        pl.kernel,
        out_type=jax.ShapeDtypeStruct((_B * _N, _C), jnp.float32),
        mesh=mesh,
        scratch_types=[
            pltpu.VMEM((_N, _C), jnp.float32),
            pltpu.VMEM((_K, _N), jnp.int32),
            pltpu.VMEM((_L, _C), jnp.float32),
        ],
        compiler_params=pltpu.CompilerParams(
            use_tc_tiling_on_sc=False, needs_layout_passes=False),
    )(_sc_raw_body)


def _tc_body(x_ref, xjm_ref, w1_ref, b1_ref, w2a_ref, w2b_ref, b2_ref, out_ref):
    x = x_ref[0]
    h = jnp.maximum(
        lax.dot_general(w1_ref[...], x, (((0,), (0,)), ((), ())),
                        preferred_element_type=jnp.float32) + b1_ref[...],
        0.0)
    xjm = xjm_ref[0]
    pre = (
        lax.dot_general(w2a_ref[...], h, (((0,), (0,)), ((), ())),
                        preferred_element_type=jnp.float32)
        + lax.dot_general(w2b_ref[...], xjm, (((0,), (1,)), ((), ())),
                          preferred_element_type=jnp.float32)
        + b2_ref[...])
    out_ref[0] = jax.nn.sigmoid(jnp.maximum(pre, 0.0))


_tc_dense = pl.pallas_call(
    _tc_body,
    grid=(_B,),
    in_specs=[
        pl.BlockSpec((1, _C, _N), lambda b: (b, 0, 0)),
        pl.BlockSpec((1, _N, _C), lambda b: (b, 0, 0)),
        pl.BlockSpec((_C, _C), lambda b: (0, 0)),
        pl.BlockSpec((_C, 1), lambda b: (0, 0)),
        pl.BlockSpec((_C, _C_OUT), lambda b: (0, 0)),
        pl.BlockSpec((_C, _C_OUT), lambda b: (0, 0)),
        pl.BlockSpec((_C_OUT, 1), lambda b: (0, 0)),
    ],
    out_specs=pl.BlockSpec((1, _C_OUT, _N), lambda b: (b, 0, 0)),
    out_shape=jax.ShapeDtypeStruct((_B, _C_OUT, _N), jnp.float32),
)


def kernel(x, edge_index, W1, b1, W2, b2):
    x_sq = x[:, :, :, 0]
    x_t = jnp.transpose(x_sq, (0, 2, 1)).reshape(_B * _N, _C)
    idx_t = jnp.transpose(edge_index[0].astype(jnp.int32), (0, 2, 1))
    xjm = _sc_gather_max()(x_t, idx_t)
    out = _tc_dense(x_sq, xjm.reshape(_B, _N, _C), W1,
                    b1.reshape(_C, 1), W2[:_C], W2[_C:],
                    b2.reshape(_C_OUT, 1))
    return out[:, :, :, None]


# speedup vs baseline: 8.8950x; 3.4847x over previous
import functools

import jax
import jax.numpy as jnp
from jax import lax
from jax.experimental import pallas as pl
from jax.experimental.pallas import tpu as pltpu
from jax.experimental.pallas import tpu_sc as plsc

_B, _C, _N, _K = 32, 384, 256, 16
_C_OUT = 768
_L = 16

_NCHUNK = 64
_JC = 2
_JV = _C // _L // _JC


def _sc_raw_body(x_hbm, idx_hbm, out_hbm, table_v, out_v, idx_v):
    cid = lax.axis_index("c")
    sid = lax.axis_index("s")
    wid = sid * 2 + cid

    pltpu.sync_copy(x_hbm.at[wid], table_v)
    pltpu.sync_copy(idx_hbm.at[wid], idx_v)

    lanes = lax.iota(jnp.int32, _L)

    def chunk_body(ch, _):
        node0 = ch * _NCHUNK

        def node_body(i, _):
            ioff = (node0 + i) * _K
            for jc in range(_JC):
                acc = [None] * _JV
                for r in range(_K):
                    rid = plsc.load_gather(
                        idx_v, [jnp.full((_L,), 0, jnp.int32) + (ioff + r)])
                    base = rid * _C + lanes
                    for j in range(_JV):
                        off = (jc * _JV + j) * _L
                        v = plsc.load_gather(table_v, [base + off])
                        acc[j] = v if r == 0 else jnp.maximum(acc[j], v)
                for j in range(_JV):
                    out_v[i, pl.ds((jc * _JV + j) * _L, _L)] = acc[j]
            return ()

        lax.fori_loop(0, _NCHUNK, node_body, ())
        pltpu.sync_copy(out_v, out_hbm.at[pl.ds(wid * _N + node0, _NCHUNK)])
        return ()

    lax.fori_loop(0, _N // _NCHUNK, chunk_body, ())


@functools.cache
def _sc_gather_max():
    mesh = plsc.VectorSubcoreMesh(
        core_axis_name="c", subcore_axis_name="s",
        num_cores=2, num_subcores=16)
    return functools.partial(
        pl.kernel,
        out_type=jax.ShapeDtypeStruct((_B * _N, _C), jnp.float32),
        mesh=mesh,
        scratch_types=[
            pltpu.VMEM((_N * _C,), jnp.float32),
            pltpu.VMEM((_NCHUNK, _C), jnp.float32),
            pltpu.VMEM((_N * _K,), jnp.int32),
        ],
        compiler_params=pltpu.CompilerParams(
            use_tc_tiling_on_sc=False, needs_layout_passes=False),
    )(_sc_raw_body)


def _tc_body(x_ref, xjm_ref, w1_ref, b1_ref, w2a_ref, w2b_ref, b2_ref, out_ref):
    x = x_ref[0]
    h = jnp.maximum(
        lax.dot_general(w1_ref[...], x, (((0,), (0,)), ((), ())),
                        preferred_element_type=jnp.float32) + b1_ref[...],
        0.0)
    xjm = xjm_ref[0]
    pre = (
        lax.dot_general(w2a_ref[...], h, (((0,), (0,)), ((), ())),
                        preferred_element_type=jnp.float32)
        + lax.dot_general(w2b_ref[...], xjm, (((0,), (1,)), ((), ())),
                          preferred_element_type=jnp.float32)
        + b2_ref[...])
    out_ref[0] = jax.nn.sigmoid(jnp.maximum(pre, 0.0))


_tc_dense = pl.pallas_call(
    _tc_body,
    grid=(_B,),
    in_specs=[
        pl.BlockSpec((1, _C, _N), lambda b: (b, 0, 0)),
        pl.BlockSpec((1, _N, _C), lambda b: (b, 0, 0)),
        pl.BlockSpec((_C, _C), lambda b: (0, 0)),
        pl.BlockSpec((_C, 1), lambda b: (0, 0)),
        pl.BlockSpec((_C, _C_OUT), lambda b: (0, 0)),
        pl.BlockSpec((_C, _C_OUT), lambda b: (0, 0)),
        pl.BlockSpec((_C_OUT, 1), lambda b: (0, 0)),
    ],
    out_specs=pl.BlockSpec((1, _C_OUT, _N), lambda b: (b, 0, 0)),
    out_shape=jax.ShapeDtypeStruct((_B, _C_OUT, _N), jnp.float32),
)


def kernel(x, edge_index, W1, b1, W2, b2):
    x_sq = x[:, :, :, 0]
    x_t = jnp.transpose(x_sq, (0, 2, 1)).reshape(_B, _N * _C)
    idx = edge_index[0].astype(jnp.int32).reshape(_B, _N * _K)
    xjm = _sc_gather_max()(x_t, idx)
    out = _tc_dense(x_sq, xjm.reshape(_B, _N, _C), W1,
                    b1.reshape(_C, 1), W2[:_C], W2[_C:],
                    b2.reshape(_C_OUT, 1))
    return out[:, :, :, None]
